# baseline (speedup 1.0000x reference)
import jax
import jax.numpy as jnp
from jax.experimental import pallas as pl
from jax.experimental.pallas import tpu as pltpu

_BLOCK = 4096


def _rq_body(x_ref, pi_ref, cb_ref, out_ref):
    xb = x_ref[:]
    norm = jnp.sqrt(jnp.sum(xb * xb, axis=1, keepdims=True))
    xn = xb / (norm + 1e-8)
    xr = jax.lax.dot_general(xn, pi_ref[:], (((1,), (1,)), ((), ())),
                             preferred_element_type=jnp.float32)

    k = cb_ref.shape[1]
    c0 = cb_ref[0, 0]
    step = (cb_ref[0, k - 1] - c0) / (k - 1)
    idx = jnp.clip(jnp.round((xr - c0) / step), 0.0, float(k - 1))
    xq = c0 + idx * step
    res = xr - xq
    scale = jnp.mean(jnp.abs(res), axis=1, keepdims=True)
    xc = xq + jnp.where(res >= 0.0, scale, -scale)

    out = jnp.dot(xc, pi_ref[:], preferred_element_type=jnp.float32)
    out_ref[:] = out * norm


def kernel(x, Pi, centroids):
    n, d = x.shape
    k = centroids.shape[0]
    cb = centroids.reshape(1, k)
    return pl.pallas_call(
        _rq_body,
        grid=(n // _BLOCK,),
        in_specs=[
            pl.BlockSpec((_BLOCK, d), lambda i: (i, 0)),
            pl.BlockSpec((d, d), lambda i: (0, 0)),
            pl.BlockSpec((1, k), lambda i: (0, 0)),
        ],
        out_specs=pl.BlockSpec((_BLOCK, d), lambda i: (i, 0)),
        out_shape=jax.ShapeDtypeStruct((n, d), x.dtype),
        compiler_params=pltpu.CompilerParams(
            dimension_semantics=("parallel",),
        ),
    )(x, Pi, cb)
